# Initial kernel scaffold; baseline (speedup 1.0000x reference)
import jax
import jax.numpy as jnp
from jax.experimental import pallas as pl


def kernel(x, edge_index, w_proj, b_proj, ggnn_w, gru_wih, gru_whh, gru_bih, gru_bhh, conv1_w, conv1_b, conv2_w, conv2_b, fc1_w, fc1_b, fc2_w, fc2_b):
    raise NotImplementedError("write your pallas kernel here")


# speedup vs baseline: 1.0655x; 1.0655x over previous
import functools

import jax
import jax.numpy as jnp
from jax import lax
from jax.experimental import pallas as pl
from jax.experimental.pallas import tpu as pltpu

N = 10000
E = 320000
HID = 128
L = 6
NB = 1000
GRID = N // NB


def _sigmoid(x):
    return 1.0 / (1.0 + jnp.exp(-x))


def _proj_body(x_ref, wpT_ref, b_ref, w0_ref, h_ref, m_ref):
    h = jnp.dot(x_ref[...], wpT_ref[...], preferred_element_type=jnp.float32)
    h = h + b_ref[...]
    h_ref[...] = h
    m_ref[...] = jnp.dot(h, w0_ref[...], preferred_element_type=jnp.float32)


def _proj(x, wpT, b2d, w0):
    return pl.pallas_call(
        _proj_body,
        grid=(GRID,),
        in_specs=[
            pl.BlockSpec((NB, HID), lambda i: (i, 0)),
            pl.BlockSpec((HID, HID), lambda i: (0, 0)),
            pl.BlockSpec((1, HID), lambda i: (0, 0)),
            pl.BlockSpec((HID, HID), lambda i: (0, 0)),
        ],
        out_specs=[
            pl.BlockSpec((NB, HID), lambda i: (i, 0)),
            pl.BlockSpec((NB, HID), lambda i: (i, 0)),
        ],
        out_shape=[
            jax.ShapeDtypeStruct((N, HID), jnp.float32),
            jax.ShapeDtypeStruct((N, HID), jnp.float32),
        ],
    )(x, wpT, b2d, w0)


def _gru_body(h_ref, p0_ref, p1_ref, wihT_ref, whhT_ref, bih_ref, bhh_ref,
              wn_ref, hn_ref, mn_ref):
    h = h_ref[...]
    agg = p0_ref[...] + p1_ref[...]
    gi = jnp.dot(agg, wihT_ref[...], preferred_element_type=jnp.float32)
    gi = gi + bih_ref[...]
    gh = jnp.dot(h, whhT_ref[...], preferred_element_type=jnp.float32)
    gh = gh + bhh_ref[...]
    r = _sigmoid(gi[:, :HID] + gh[:, :HID])
    z = _sigmoid(gi[:, HID:2 * HID] + gh[:, HID:2 * HID])
    n = jnp.tanh(gi[:, 2 * HID:] + r * gh[:, 2 * HID:])
    hn = (1.0 - z) * n + z * h
    hn_ref[...] = hn
    mn_ref[...] = jnp.dot(hn, wn_ref[...], preferred_element_type=jnp.float32)


def _gru_step(h, p0, p1, wihT, whhT, bih2d, bhh2d, wn):
    return pl.pallas_call(
        _gru_body,
        grid=(GRID,),
        in_specs=[
            pl.BlockSpec((NB, HID), lambda i: (i, 0)),
            pl.BlockSpec((NB, HID), lambda i: (i, 0)),
            pl.BlockSpec((NB, HID), lambda i: (i, 0)),
            pl.BlockSpec((HID, 3 * HID), lambda i: (0, 0)),
            pl.BlockSpec((HID, 3 * HID), lambda i: (0, 0)),
            pl.BlockSpec((1, 3 * HID), lambda i: (0, 0)),
            pl.BlockSpec((1, 3 * HID), lambda i: (0, 0)),
            pl.BlockSpec((HID, HID), lambda i: (0, 0)),
        ],
        out_specs=[
            pl.BlockSpec((NB, HID), lambda i: (i, 0)),
            pl.BlockSpec((NB, HID), lambda i: (i, 0)),
        ],
        out_shape=[
            jax.ShapeDtypeStruct((N, HID), jnp.float32),
            jax.ShapeDtypeStruct((N, HID), jnp.float32),
        ],
    )(h, p0, p1, wihT, whhT, bih2d, bhh2d, wn)


def _scatter_add(m, src, dst):
    agg = jnp.zeros((N, HID), jnp.float32).at[dst].add(m[src])
    return agg, jnp.zeros((N, HID), jnp.float32)


def _conv1d(x, w, b, pad):
    out = lax.conv_general_dilated(
        x, w, window_strides=(1,), padding=[(pad, pad)],
        dimension_numbers=('NCH', 'OIH', 'NCH'))
    return out + b[None, :, None]


def _maxpool1d(x, k, s):
    return lax.reduce_window(x, -jnp.inf, lax.max, (1, 1, k), (1, 1, s),
                             'VALID')


def _readout(h, x, conv1_w, conv1_b, conv2_w, conv2_b, fc1_w, fc1_b, fc2_w,
             fc2_b):
    concat = jnp.concatenate([h, x], axis=1)[:, None, :]
    Z = _maxpool1d(jax.nn.relu(_conv1d(concat, conv1_w, conv1_b, 1)), 3, 2)
    Z = _maxpool1d(_conv1d(Z, conv2_w, conv2_b, 1), 2, 2)
    hh = h[:, None, :]
    Y = _maxpool1d(jax.nn.relu(_conv1d(hh, conv1_w, conv1_b, 1)), 3, 2)
    Y = _maxpool1d(_conv1d(Y, conv2_w, conv2_b, 1), 2, 2)
    Zf = Z.reshape(N, -1)
    Yf = Y.reshape(N, -1)
    res = (Zf @ fc1_w.T + fc1_b) * (Yf @ fc2_w.T + fc2_b)
    p = jax.nn.sigmoid(res.reshape(-1))
    eps = 1e-6
    p = jnp.clip(p, eps, 1.0 - eps)
    z1 = jnp.log(p / (1.0 - p))
    return jnp.stack([jnp.zeros_like(z1), z1], axis=1)


def kernel(x, edge_index, w_proj, b_proj, ggnn_w, gru_wih, gru_whh, gru_bih,
           gru_bhh, conv1_w, conv1_b, conv2_w, conv2_b, fc1_w, fc1_b, fc2_w,
           fc2_b):
    src = edge_index[0]
    dst = edge_index[1]
    wpT = w_proj.T
    b2d = b_proj.reshape(1, HID)
    wihT = gru_wih.T
    whhT = gru_whh.T
    bih2d = gru_bih.reshape(1, 3 * HID)
    bhh2d = gru_bhh.reshape(1, 3 * HID)

    h, m = _proj(x, wpT, b2d, ggnn_w[0])
    for l in range(L):
        p0, p1 = _scatter_add(m, src, dst)
        wn = ggnn_w[l + 1] if l + 1 < L else ggnn_w[0]
        h, m = _gru_step(h, p0, p1, wihT, whhT, bih2d, bhh2d, wn)

    return _readout(h, x, conv1_w, conv1_b, conv2_w, conv2_b, fc1_w, fc1_b,
                    fc2_w, fc2_b)
